# baseline (speedup 1.0000x reference)
import functools

import jax
import jax.numpy as jnp
from jax import lax
from jax.experimental import pallas as pl
from jax.experimental.pallas import tpu as pltpu
from jax.experimental.pallas import tpu_sc as plsc

NB = 8
P = 2048
K = 20
N = NB * P
E = N * K
EPS = 1e-5
BF = jnp.bfloat16


def _bn_stats_apply(x, s, q, n, g, be):
    m = s / n
    v = jnp.maximum(q / n - m * m, 0.0)
    return g * (x - m) / jnp.sqrt(v + EPS) + be


def _prep0_body(pos_ref, posp_ref, x16_ref, n2_ref):
    x = pos_ref[...]
    r = x.shape[0]
    posp_ref[...] = jnp.concatenate(
        [x, jnp.zeros((r, 13), jnp.float32)], axis=1)
    x16_ref[...] = jnp.concatenate(
        [x, jnp.zeros((r, 5), jnp.float32)], axis=1).astype(BF)
    n2_ref[...] = jnp.sum(x * x, axis=1, keepdims=True)


def _prep0(pos):
    rb = P
    return pl.pallas_call(
        _prep0_body,
        grid=(N // rb,),
        in_specs=[pl.BlockSpec((rb, 3), lambda g: (g, 0))],
        out_specs=[
            pl.BlockSpec((rb, 16), lambda g: (g, 0)),
            pl.BlockSpec((rb, 8), lambda g: (g, 0)),
            pl.BlockSpec((rb, 1), lambda g: (g, 0)),
        ],
        out_shape=[
            jax.ShapeDtypeStruct((N, 16), jnp.float32),
            jax.ShapeDtypeStruct((N, 8), BF),
            jax.ShapeDtypeStruct((N, 1), jnp.float32),
        ],
    )(pos)


def _knn_body(xr_ref, xc_ref, n2r_ref, n2c_ref, out_ref):
    xr = xr_ref[...]
    xc = xc_ref[...]
    r = xr.shape[0]
    cross = lax.dot_general(xr, xc, (((1,), (1,)), ((), ())),
                            preferred_element_type=jnp.float32)
    d = (n2r_ref[...] + n2c_ref[...]) - 2.0 * cross
    ii = lax.broadcasted_iota(jnp.int32, (r, P), 1)
    big = jnp.float32(3.0e38)
    cols = []
    for _ in range(K):
        m = jnp.min(d, axis=1, keepdims=True)
        j = jnp.min(jnp.where(d == m, ii, P), axis=1, keepdims=True)
        cols.append(j)
        d = jnp.where(ii == j, big, d)
    idx = jnp.concatenate(cols, axis=1)
    out_ref[...] = jnp.transpose(idx) + pl.program_id(0) * P


def _knn(x16, n2, n2row, dp, rb=512):
    return pl.pallas_call(
        _knn_body,
        grid=(NB, P // rb),
        in_specs=[
            pl.BlockSpec((rb, dp), lambda c, g: (c * (P // rb) + g, 0)),
            pl.BlockSpec((P, dp), lambda c, g: (c, 0)),
            pl.BlockSpec((rb, 1), lambda c, g: (c * (P // rb) + g, 0)),
            pl.BlockSpec((1, P), lambda c, g: (0, c)),
        ],
        out_specs=pl.BlockSpec((K, rb), lambda c, g: (0, c * (P // rb) + g)),
        out_shape=jax.ShapeDtypeStruct((K, N), jnp.int32),
    )(x16, x16, n2, n2row)


def _make_sc_gather(d, ch):
    info = plsc.get_sparse_core_info()
    nw = info.num_cores * info.num_subcores
    bpw = E // nw
    nchunk = bpw // ch
    assert bpw % ch == 0
    mesh = plsc.VectorSubcoreMesh(core_axis_name="c", subcore_axis_name="s")

    @functools.partial(
        pl.kernel,
        mesh=mesh,
        out_type=jax.ShapeDtypeStruct((E, d), jnp.float32),
        compiler_params=pltpu.CompilerParams(use_tc_tiling_on_sc=False,
                                             skip_device_barrier=True),
        scratch_types=[
            pltpu.VMEM((bpw,), jnp.int32),
            pltpu.VMEM((ch, d), jnp.float32),
            pltpu.VMEM((ch, d), jnp.float32),
            pltpu.SemaphoreType.DMA,
            pltpu.SemaphoreType.DMA,
            pltpu.SemaphoreType.DMA,
            pltpu.SemaphoreType.DMA,
        ],
    )
    def gk(table_hbm, idx_hbm, out_hbm, idx_v, rows0, rows1, g0, g1, w0, w1):
        wid = lax.axis_index("s") * info.num_cores + lax.axis_index("c")
        base = wid * bpw
        rows = (rows0, rows1)
        gsem = (g0, g1)
        wsem = (w0, w1)
        pltpu.sync_copy(idx_hbm.at[pl.ds(base, bpw)], idx_v)
        pltpu.async_copy(table_hbm.at[idx_v.at[pl.ds(0, ch)]], rows0, g0)
        for i in range(nchunk):
            b = i % 2
            nb = (i + 1) % 2
            if i + 1 < nchunk:
                if i >= 1:
                    pltpu.make_async_copy(rows[nb], out_hbm.at[pl.ds(0, ch)],
                                          wsem[nb]).wait()
                pltpu.async_copy(table_hbm.at[idx_v.at[pl.ds((i + 1) * ch, ch)]],
                                 rows[nb], gsem[nb])
            pltpu.make_async_copy(table_hbm.at[idx_v.at[pl.ds(i * ch, ch)]],
                                  rows[b], gsem[b]).wait()
            pltpu.async_copy(rows[b], out_hbm.at[pl.ds(base + i * ch, ch)],
                             wsem[b])
        for i in (nchunk - 2, nchunk - 1):
            if i >= 0:
                pltpu.make_async_copy(rows[i % 2],
                                      out_hbm.at[pl.ds(base + i * ch, ch)],
                                      wsem[i % 2]).wait()

    return gk


def _gather_rows(table, idx_flat, d, ch):
    return _make_sc_gather(d, ch)(table, idx_flat)


def _ec1_body(nlayers, g_ref, xi_ref, w1_ref, b1_ref, m1_ref, sq1_ref,
              g1_ref, be1_ref, w2_ref, b2_ref, m2_ref, sq2_ref, g2_ref,
              be2_ref, w3_ref, b3_ref, s_ref, q_ref, mx_ref=None,
              mn_ref=None):
    xi = xi_ref[...]
    rb = xi.shape[0]
    xi3 = xi[:, :3]
    z2 = jnp.zeros((rb, 2), jnp.float32)
    tot_s = jnp.zeros((1, 64), jnp.float32)
    tot_q = jnp.zeros((1, 64), jnp.float32)
    mx = None
    mn = None
    for k in range(K):
        xj = g_ref[k]
        dlt = (xj - xi)[:, :3]
        h0 = jnp.concatenate([xi3, dlt, z2], axis=1).astype(BF)
        h = jnp.maximum(
            jnp.dot(h0, w1_ref[...], preferred_element_type=jnp.float32)
            + b1_ref[...], 0.0)
        if nlayers >= 2:
            hn = _bn_stats_apply(h, m1_ref[...], sq1_ref[...], float(E),
                                 g1_ref[...], be1_ref[...]).astype(BF)
            h = jnp.maximum(
                jnp.dot(hn, w2_ref[...], preferred_element_type=jnp.float32)
                + b2_ref[...], 0.0)
        if nlayers >= 3:
            hn = _bn_stats_apply(h, m2_ref[...], sq2_ref[...], float(E),
                                 g2_ref[...], be2_ref[...]).astype(BF)
            h = jnp.maximum(
                jnp.dot(hn, w3_ref[...], preferred_element_type=jnp.float32)
                + b3_ref[...], 0.0)
        tot_s += jnp.sum(h, axis=0, keepdims=True)
        tot_q += jnp.sum(h * h, axis=0, keepdims=True)
        if mx_ref is not None:
            mx = h if mx is None else jnp.maximum(mx, h)
            mn = h if mn is None else jnp.minimum(mn, h)

    @pl.when(pl.program_id(0) == 0)
    def _():
        s_ref[...] = jnp.zeros_like(s_ref)
        q_ref[...] = jnp.zeros_like(q_ref)

    s_ref[...] += tot_s
    q_ref[...] += tot_q
    if mx_ref is not None:
        mx_ref[...] = mx
        mn_ref[...] = mn


def _ec1(nlayers, g1, posp, w116, b1, bn1, w216, b2, bn2, w316, b3, rb=512):
    ng = N // rb
    stat = jax.ShapeDtypeStruct((1, 64), jnp.float32)
    vspec = pl.BlockSpec((1, 64), lambda g: (0, 0))
    out_shape = [stat, stat]
    out_specs = [vspec, vspec]
    if nlayers == 3:
        out_shape += [jax.ShapeDtypeStruct((N, 64), jnp.float32)] * 2
        out_specs += [pl.BlockSpec((rb, 64), lambda g: (g, 0))] * 2
    return pl.pallas_call(
        functools.partial(_ec1_body, nlayers),
        grid=(ng,),
        in_specs=[
            pl.BlockSpec((K, rb, 16), lambda g: (0, g, 0)),
            pl.BlockSpec((rb, 16), lambda g: (g, 0)),
            pl.BlockSpec((8, 64), lambda g: (0, 0)),
            vspec, vspec, vspec, vspec, vspec,
            pl.BlockSpec((64, 64), lambda g: (0, 0)),
            vspec, vspec, vspec, vspec, vspec,
            pl.BlockSpec((64, 64), lambda g: (0, 0)),
            vspec,
        ],
        out_specs=out_specs,
        out_shape=out_shape,
    )(g1, posp, w116, b1, *bn1, w216, b2, *bn2, w316, b3)


def _prep1_body(mx_ref, mn_ref, m_ref, sq_ref, g_ref, be_ref,
                x1_ref, x16_ref, n2_ref):
    g = g_ref[...]
    sel = jnp.where(g > 0, mx_ref[...], mn_ref[...])
    x1 = _bn_stats_apply(sel, m_ref[...], sq_ref[...], float(E), g,
                         be_ref[...])
    x1_ref[...] = x1
    x16_ref[...] = x1.astype(BF)
    n2_ref[...] = jnp.sum(x1 * x1, axis=1, keepdims=True)


def _prep1(m3x, m3n, bn3):
    rb = P
    vspec = pl.BlockSpec((1, 64), lambda g: (0, 0))
    return pl.pallas_call(
        _prep1_body,
        grid=(N // rb,),
        in_specs=[
            pl.BlockSpec((rb, 64), lambda g: (g, 0)),
            pl.BlockSpec((rb, 64), lambda g: (g, 0)),
            vspec, vspec, vspec, vspec,
        ],
        out_specs=[
            pl.BlockSpec((rb, 64), lambda g: (g, 0)),
            pl.BlockSpec((rb, 64), lambda g: (g, 0)),
            pl.BlockSpec((rb, 1), lambda g: (g, 0)),
        ],
        out_shape=[
            jax.ShapeDtypeStruct((N, 64), jnp.float32),
            jax.ShapeDtypeStruct((N, 64), BF),
            jax.ShapeDtypeStruct((N, 1), jnp.float32),
        ],
    )(m3x, m3n, bn3[0], bn3[1], bn3[2], bn3[3])


def _ec2_body(g_ref, xi_ref, w_ref, b_ref, s_ref, q_ref, mx_ref, mn_ref):
    xi = xi_ref[...]
    tot_s = jnp.zeros((1, 128), jnp.float32)
    tot_q = jnp.zeros((1, 128), jnp.float32)
    mx = None
    mn = None
    for k in range(K):
        xj = g_ref[k]
        h0 = jnp.concatenate([xi, xj - xi], axis=1).astype(BF)
        h = jnp.maximum(
            jnp.dot(h0, w_ref[...], preferred_element_type=jnp.float32)
            + b_ref[...], 0.0)
        tot_s += jnp.sum(h, axis=0, keepdims=True)
        tot_q += jnp.sum(h * h, axis=0, keepdims=True)
        mx = h if mx is None else jnp.maximum(mx, h)
        mn = h if mn is None else jnp.minimum(mn, h)

    @pl.when(pl.program_id(0) == 0)
    def _():
        s_ref[...] = jnp.zeros_like(s_ref)
        q_ref[...] = jnp.zeros_like(q_ref)

    s_ref[...] += tot_s
    q_ref[...] += tot_q
    mx_ref[...] = mx
    mn_ref[...] = mn


def _ec2(g2, x1, w16, b, rb=256):
    stat = jax.ShapeDtypeStruct((1, 128), jnp.float32)
    return pl.pallas_call(
        _ec2_body,
        grid=(N // rb,),
        in_specs=[
            pl.BlockSpec((K, rb, 64), lambda g: (0, g, 0)),
            pl.BlockSpec((rb, 64), lambda g: (g, 0)),
            pl.BlockSpec((128, 128), lambda g: (0, 0)),
            pl.BlockSpec((1, 128), lambda g: (0, 0)),
        ],
        out_specs=[
            pl.BlockSpec((1, 128), lambda g: (0, 0)),
            pl.BlockSpec((1, 128), lambda g: (0, 0)),
            pl.BlockSpec((rb, 128), lambda g: (g, 0)),
            pl.BlockSpec((rb, 128), lambda g: (g, 0)),
        ],
        out_shape=[stat, stat,
                   jax.ShapeDtypeStruct((N, 128), jnp.float32),
                   jax.ShapeDtypeStruct((N, 128), jnp.float32)],
    )(g2, x1, w16, b)


def _l1_body(x1_ref, mx2_ref, mn2_ref, m_ref, sq_ref, g_ref, be_ref,
             w_ref, b_ref, s_ref, q_ref, mx_ref, mn_ref):
    g = g_ref[...]
    sel = jnp.where(g > 0, mx2_ref[...], mn2_ref[...])
    x2 = _bn_stats_apply(sel, m_ref[...], sq_ref[...], float(E), g,
                         be_ref[...])
    h = jnp.concatenate([x1_ref[...], x2], axis=1).astype(BF)
    a = jnp.maximum(
        jnp.dot(h, w_ref[...], preferred_element_type=jnp.float32)
        + b_ref[...], 0.0)
    bsum = jnp.sum(a, axis=0, keepdims=True)
    bsq = jnp.sum(a * a, axis=0, keepdims=True)
    bmax3 = jnp.max(a, axis=0, keepdims=True).reshape(1, 1, 1024)
    bmin3 = jnp.min(a, axis=0, keepdims=True).reshape(1, 1, 1024)
    c = pl.program_id(0)
    r = pl.program_id(1)

    @pl.when(jnp.logical_and(c == 0, r == 0))
    def _():
        s_ref[...] = jnp.zeros_like(s_ref)
        q_ref[...] = jnp.zeros_like(q_ref)

    s_ref[...] += bsum
    q_ref[...] += bsq

    @pl.when(r == 0)
    def _():
        mx_ref[...] = bmax3
        mn_ref[...] = bmin3

    @pl.when(r > 0)
    def _():
        mx_ref[...] = jnp.maximum(mx_ref[...], bmax3)
        mn_ref[...] = jnp.minimum(mn_ref[...], bmin3)


def _l1(x1, mx2, mn2, bnc2, w16, b, rb=256):
    stat = jax.ShapeDtypeStruct((1, 1024), jnp.float32)
    v128 = pl.BlockSpec((1, 128), lambda c, g: (0, 0))
    return pl.pallas_call(
        _l1_body,
        grid=(NB, P // rb),
        in_specs=[
            pl.BlockSpec((rb, 64), lambda c, g: (c * (P // rb) + g, 0)),
            pl.BlockSpec((rb, 128), lambda c, g: (c * (P // rb) + g, 0)),
            pl.BlockSpec((rb, 128), lambda c, g: (c * (P // rb) + g, 0)),
            v128, v128, v128, v128,
            pl.BlockSpec((192, 1024), lambda c, g: (0, 0)),
            pl.BlockSpec((1, 1024), lambda c, g: (0, 0)),
        ],
        out_specs=[
            pl.BlockSpec((1, 1024), lambda c, g: (0, 0)),
            pl.BlockSpec((1, 1024), lambda c, g: (0, 0)),
            pl.BlockSpec((1, 1, 1024), lambda c, g: (c, 0, 0)),
            pl.BlockSpec((1, 1, 1024), lambda c, g: (c, 0, 0)),
        ],
        out_shape=[stat, stat,
                   jax.ShapeDtypeStruct((NB, 1, 1024), jnp.float32),
                   jax.ShapeDtypeStruct((NB, 1, 1024), jnp.float32)],
    )(x1, mx2, mn2, bnc2[0], bnc2[1], bnc2[2], bnc2[3], w16, b)


def _head_body(mx_ref, mn_ref, m_ref, sq_ref, g_ref, be_ref,
               m1w_ref, m1b_ref, m1g_ref, m1be_ref,
               m2w_ref, m2b_ref, m2g_ref, m2be_ref,
               m3w_ref, m3b_ref, out_ref):
    g = g_ref[...]
    sel = jnp.where(g > 0, mx_ref[...], mn_ref[...])
    pooled = _bn_stats_apply(sel, m_ref[...], sq_ref[...], float(N), g,
                             be_ref[...])

    def blk(x, w, bb, gg, be):
        a = jnp.maximum(
            jnp.dot(x.astype(BF), w, preferred_element_type=jnp.float32)
            + bb, 0.0)
        m = jnp.mean(a, axis=0, keepdims=True)
        v = jnp.mean((a - m) * (a - m), axis=0, keepdims=True)
        return gg * (a - m) / jnp.sqrt(v + EPS) + be

    h = blk(pooled, m1w_ref[...], m1b_ref[...], m1g_ref[...], m1be_ref[...])
    h = blk(h, m2w_ref[...], m2b_ref[...], m2g_ref[...], m2be_ref[...])
    logits = jnp.dot(h.astype(BF), m3w_ref[...],
                     preferred_element_type=jnp.float32) + m3b_ref[...]
    zmax = jnp.max(logits, axis=1, keepdims=True)
    shifted = logits - zmax
    lse = jnp.log(jnp.sum(jnp.exp(shifted), axis=1, keepdims=True))
    out_ref[...] = shifted - lse


def _head(mx, mn, bnl, m1w, m1b, m1g, m1be, m2w, m2b, m2g, m2be, m3w, m3b):
    full = lambda s: pl.BlockSpec(s, lambda: (0,) * len(s))
    return pl.pallas_call(
        _head_body,
        grid=(),
        in_specs=[
            full((NB, 1024)), full((NB, 1024)),
            full((1, 1024)), full((1, 1024)), full((1, 1024)),
            full((1, 1024)),
            full((1024, 512)), full((1, 512)), full((1, 512)), full((1, 512)),
            full((512, 256)), full((1, 256)), full((1, 256)), full((1, 256)),
            full((256, 40)), full((1, 40)),
        ],
        out_specs=full((NB, 40)),
        out_shape=jax.ShapeDtypeStruct((NB, 40), jnp.float32),
    )(mx, mn, bnl[0], bnl[1], bnl[2], bnl[3],
      m1w, m1b, m1g, m1be, m2w, m2b, m2g, m2be, m3w, m3b)


def _row(x):
    return x.reshape(1, -1)


def kernel(pos, batch, c1l1w, c1l1b, c1l1g, c1l1be, c1l2w, c1l2b, c1l2g,
           c1l2be, c1l3w, c1l3b, c1l3g, c1l3be, c2l1w, c2l1b, c2l1g, c2l1be,
           l1w, l1b, l1g, l1be, m1w, m1b, m1g, m1be, m2w, m2b, m2g, m2be,
           m3w, m3b):
    f32 = jnp.float32
    ne = float(E)

    w116 = jnp.zeros((8, 64), f32).at[:6].set(c1l1w).astype(BF)
    w216 = c1l2w.astype(BF)
    w316 = c1l3w.astype(BF)
    wc16 = c2l1w.astype(BF)
    wl16 = l1w.astype(BF)

    zb = (_row(jnp.zeros(64, f32)),) * 4

    posp, p16, n2a = _prep0(pos)
    n2arow = n2a.reshape(N)[None, :]

    idx1 = _knn(p16, n2a, n2arow, 8)
    g1 = _gather_rows(posp, idx1.reshape(-1), 16, 2048).reshape(K, N, 16)

    s1, q1 = _ec1(1, g1, posp, w116, _row(c1l1b), zb, w216, _row(c1l2b),
                  zb, w316, _row(c1l3b))[:2]
    bn1 = (s1, q1, _row(c1l1g), _row(c1l1be))
    s2, q2 = _ec1(2, g1, posp, w116, _row(c1l1b), bn1, w216, _row(c1l2b),
                  zb, w316, _row(c1l3b))[:2]
    bn2 = (s2, q2, _row(c1l2g), _row(c1l2be))
    s3, q3, m3x, m3n = _ec1(3, g1, posp, w116, _row(c1l1b), bn1, w216,
                            _row(c1l2b), bn2, w316, _row(c1l3b))
    bn3 = (s3, q3, _row(c1l3g), _row(c1l3be))

    x1, x116, n2b = _prep1(m3x, m3n, bn3)
    n2brow = n2b.reshape(N)[None, :]

    idx2 = _knn(x116, n2b, n2brow, 64)
    g2 = _gather_rows(x1, idx2.reshape(-1), 64, 640).reshape(K, N, 64)

    sc_, qc_, m2x, m2n = _ec2(g2, x1, wc16, _row(c2l1b))
    bnc2 = (sc_, qc_, _row(c2l1g), _row(c2l1be))

    sl_, ql_, mxp, mnp = _l1(x1, m2x, m2n, bnc2, wl16, _row(l1b))
    mxp = mxp.reshape(NB, 1024)
    mnp = mnp.reshape(NB, 1024)
    bnl = (sl_, ql_, _row(l1g), _row(l1be))

    return _head(mxp, mnp, bnl,
                 m1w.astype(BF), _row(m1b), _row(m1g), _row(m1be),
                 m2w.astype(BF), _row(m2b), _row(m2g), _row(m2be),
                 m3w.astype(BF), _row(m3b))


# speedup vs baseline: 1.2242x; 1.2242x over previous
import functools

import jax
import jax.numpy as jnp
from jax import lax
from jax.experimental import pallas as pl
from jax.experimental.pallas import tpu as pltpu
from jax.experimental.pallas import tpu_sc as plsc

NB = 8
P = 2048
K = 20
N = NB * P
E = N * K
EPS = 1e-5
BF = jnp.bfloat16


def _bn_stats_apply(x, s, q, n, g, be):
    m = s / n
    v = jnp.maximum(q / n - m * m, 0.0)
    return g * (x - m) / jnp.sqrt(v + EPS) + be


def _prep0_body(pos_ref, posp_ref, x16_ref, n2_ref):
    x = pos_ref[...]
    r = x.shape[0]
    posp_ref[...] = jnp.concatenate(
        [x, jnp.zeros((r, 13), jnp.float32)], axis=1)
    x16_ref[...] = jnp.concatenate(
        [x, jnp.zeros((r, 5), jnp.float32)], axis=1).astype(BF)
    n2_ref[...] = jnp.sum(x * x, axis=1, keepdims=True)


def _prep0(pos):
    rb = P
    return pl.pallas_call(
        _prep0_body,
        grid=(N // rb,),
        in_specs=[pl.BlockSpec((rb, 3), lambda g: (g, 0))],
        out_specs=[
            pl.BlockSpec((rb, 16), lambda g: (g, 0)),
            pl.BlockSpec((rb, 8), lambda g: (g, 0)),
            pl.BlockSpec((rb, 1), lambda g: (g, 0)),
        ],
        out_shape=[
            jax.ShapeDtypeStruct((N, 16), jnp.float32),
            jax.ShapeDtypeStruct((N, 8), BF),
            jax.ShapeDtypeStruct((N, 1), jnp.float32),
        ],
    )(pos)


def _knn_body(xr_ref, xc_ref, n2r_ref, n2c_ref, out_ref):
    xr = xr_ref[...]
    xc = xc_ref[...]
    r = xr.shape[0]
    cross = lax.dot_general(xr, xc, (((1,), (1,)), ((), ())),
                            preferred_element_type=jnp.float32)
    d0 = (n2r_ref[...] + n2c_ref[...]) - 2.0 * cross
    big = jnp.float32(3.0e38)
    nv = P // 128
    m1 = 6

    pid = pl.program_id(0)
    dcs = [d0[:, c * 128:(c + 1) * 128] for c in range(nv)]
    lane = lax.broadcasted_iota(jnp.int32, (r, 128), 1)
    cvals = []
    cidx = []
    for _ in range(m1):
        lm = dcs[0]
        for c in range(1, nv):
            lm = jnp.minimum(lm, dcs[c])
        jv = jnp.full((r, 128), nv, jnp.int32)
        for c in reversed(range(nv)):
            jv = jnp.where(dcs[c] == lm, c, jv)
        for c in range(nv):
            dcs[c] = jnp.where(jv == c, big, dcs[c])
        cvals.append(lm)
        cidx.append(jv * 128 + lane)

    catv = jnp.concatenate(cvals, axis=1)
    cati = jnp.concatenate(cidx, axis=1)
    cols = []
    tau = None
    for _ in range(K):
        m = jnp.min(catv, axis=1, keepdims=True)
        j = jnp.min(jnp.where(catv == m, cati, P), axis=1, keepdims=True)
        cols.append(j)
        catv = jnp.where(cati == j, big, catv)
        tau = m
    idx = jnp.concatenate(cols, axis=1)
    out_ref[...] = jnp.transpose(idx) + pid * P

    flag = jnp.max(jnp.sum(
        (cvals[m1 - 1] <= tau).astype(jnp.int32), axis=1, keepdims=True))

    @pl.when(flag > 0)
    def _():
        d = d0
        ii = lax.broadcasted_iota(jnp.int32, (r, P), 1)
        cols2 = []
        for _ in range(K):
            m = jnp.min(d, axis=1, keepdims=True)
            j = jnp.min(jnp.where(d == m, ii, P), axis=1, keepdims=True)
            cols2.append(j)
            d = jnp.where(ii == j, big, d)
        idx2 = jnp.concatenate(cols2, axis=1)
        out_ref[...] = jnp.transpose(idx2) + pid * P


def _knn(x16, n2, n2row, dp, rb=512):
    return pl.pallas_call(
        _knn_body,
        grid=(NB, P // rb),
        in_specs=[
            pl.BlockSpec((rb, dp), lambda c, g: (c * (P // rb) + g, 0)),
            pl.BlockSpec((P, dp), lambda c, g: (c, 0)),
            pl.BlockSpec((rb, 1), lambda c, g: (c * (P // rb) + g, 0)),
            pl.BlockSpec((1, P), lambda c, g: (0, c)),
        ],
        out_specs=pl.BlockSpec((K, rb), lambda c, g: (0, c * (P // rb) + g)),
        out_shape=jax.ShapeDtypeStruct((K, N), jnp.int32),
    )(x16, x16, n2, n2row)


def _make_sc_gather(d, ch):
    info = plsc.get_sparse_core_info()
    nw = info.num_cores * info.num_subcores
    bpw = E // nw
    nchunk = bpw // ch
    assert bpw % ch == 0
    mesh = plsc.VectorSubcoreMesh(core_axis_name="c", subcore_axis_name="s")

    @functools.partial(
        pl.kernel,
        mesh=mesh,
        out_type=jax.ShapeDtypeStruct((E, d), jnp.float32),
        compiler_params=pltpu.CompilerParams(use_tc_tiling_on_sc=False),
        scratch_types=[
            pltpu.VMEM((bpw,), jnp.int32),
            pltpu.VMEM((ch, d), jnp.float32),
            pltpu.VMEM((ch, d), jnp.float32),
            pltpu.SemaphoreType.DMA,
            pltpu.SemaphoreType.DMA,
            pltpu.SemaphoreType.DMA,
            pltpu.SemaphoreType.DMA,
        ],
    )
    def gk(table_hbm, idx_hbm, out_hbm, idx_v, rows0, rows1, g0, g1, w0, w1):
        wid = lax.axis_index("s") * info.num_cores + lax.axis_index("c")
        base = wid * bpw
        rows = (rows0, rows1)
        gsem = (g0, g1)
        wsem = (w0, w1)
        pltpu.sync_copy(idx_hbm.at[pl.ds(base, bpw)], idx_v)
        pltpu.async_copy(table_hbm.at[idx_v.at[pl.ds(0, ch)]], rows0, g0)
        for i in range(nchunk):
            b = i % 2
            nb = (i + 1) % 2
            if i + 1 < nchunk:
                if i >= 1:
                    pltpu.make_async_copy(rows[nb], out_hbm.at[pl.ds(0, ch)],
                                          wsem[nb]).wait()
                pltpu.async_copy(table_hbm.at[idx_v.at[pl.ds((i + 1) * ch, ch)]],
                                 rows[nb], gsem[nb])
            pltpu.make_async_copy(table_hbm.at[idx_v.at[pl.ds(i * ch, ch)]],
                                  rows[b], gsem[b]).wait()
            pltpu.async_copy(rows[b], out_hbm.at[pl.ds(base + i * ch, ch)],
                             wsem[b])
        for i in (nchunk - 2, nchunk - 1):
            if i >= 0:
                pltpu.make_async_copy(rows[i % 2],
                                      out_hbm.at[pl.ds(base + i * ch, ch)],
                                      wsem[i % 2]).wait()

    return gk


def _gather_rows(table, idx_flat, d, ch):
    return _make_sc_gather(d, ch)(table, idx_flat)


def _ec1_body(nlayers, g_ref, xi_ref, w1_ref, b1_ref, m1_ref, sq1_ref,
              g1_ref, be1_ref, w2_ref, b2_ref, m2_ref, sq2_ref, g2_ref,
              be2_ref, w3_ref, b3_ref, s_ref, q_ref, mx_ref=None,
              mn_ref=None):
    xi = xi_ref[...]
    rb = xi.shape[0]
    xi3 = xi[:, :3]
    z2 = jnp.zeros((rb, 2), jnp.float32)
    tot_s = jnp.zeros((1, 64), jnp.float32)
    tot_q = jnp.zeros((1, 64), jnp.float32)
    mx = None
    mn = None
    for k in range(K):
        xj = g_ref[k]
        dlt = (xj - xi)[:, :3]
        h0 = jnp.concatenate([xi3, dlt, z2], axis=1).astype(BF)
        h = jnp.maximum(
            jnp.dot(h0, w1_ref[...], preferred_element_type=jnp.float32)
            + b1_ref[...], 0.0)
        if nlayers >= 2:
            hn = _bn_stats_apply(h, m1_ref[...], sq1_ref[...], float(E),
                                 g1_ref[...], be1_ref[...]).astype(BF)
            h = jnp.maximum(
                jnp.dot(hn, w2_ref[...], preferred_element_type=jnp.float32)
                + b2_ref[...], 0.0)
        if nlayers >= 3:
            hn = _bn_stats_apply(h, m2_ref[...], sq2_ref[...], float(E),
                                 g2_ref[...], be2_ref[...]).astype(BF)
            h = jnp.maximum(
                jnp.dot(hn, w3_ref[...], preferred_element_type=jnp.float32)
                + b3_ref[...], 0.0)
        tot_s += jnp.sum(h, axis=0, keepdims=True)
        tot_q += jnp.sum(h * h, axis=0, keepdims=True)
        if mx_ref is not None:
            mx = h if mx is None else jnp.maximum(mx, h)
            mn = h if mn is None else jnp.minimum(mn, h)

    @pl.when(pl.program_id(0) == 0)
    def _():
        s_ref[...] = jnp.zeros_like(s_ref)
        q_ref[...] = jnp.zeros_like(q_ref)

    s_ref[...] += tot_s
    q_ref[...] += tot_q
    if mx_ref is not None:
        mx_ref[...] = mx
        mn_ref[...] = mn


def _ec1(nlayers, g1, posp, w116, b1, bn1, w216, b2, bn2, w316, b3, rb=512):
    ng = N // rb
    stat = jax.ShapeDtypeStruct((1, 64), jnp.float32)
    vspec = pl.BlockSpec((1, 64), lambda g: (0, 0))
    out_shape = [stat, stat]
    out_specs = [vspec, vspec]
    if nlayers == 3:
        out_shape += [jax.ShapeDtypeStruct((N, 64), jnp.float32)] * 2
        out_specs += [pl.BlockSpec((rb, 64), lambda g: (g, 0))] * 2
    return pl.pallas_call(
        functools.partial(_ec1_body, nlayers),
        grid=(ng,),
        in_specs=[
            pl.BlockSpec((K, rb, 16), lambda g: (0, g, 0)),
            pl.BlockSpec((rb, 16), lambda g: (g, 0)),
            pl.BlockSpec((8, 64), lambda g: (0, 0)),
            vspec, vspec, vspec, vspec, vspec,
            pl.BlockSpec((64, 64), lambda g: (0, 0)),
            vspec, vspec, vspec, vspec, vspec,
            pl.BlockSpec((64, 64), lambda g: (0, 0)),
            vspec,
        ],
        out_specs=out_specs,
        out_shape=out_shape,
    )(g1, posp, w116, b1, *bn1, w216, b2, *bn2, w316, b3)


def _prep1_body(mx_ref, mn_ref, m_ref, sq_ref, g_ref, be_ref,
                x1_ref, x16_ref, n2_ref):
    g = g_ref[...]
    sel = jnp.where(g > 0, mx_ref[...], mn_ref[...])
    x1 = _bn_stats_apply(sel, m_ref[...], sq_ref[...], float(E), g,
                         be_ref[...])
    x1_ref[...] = x1
    x16_ref[...] = x1.astype(BF)
    n2_ref[...] = jnp.sum(x1 * x1, axis=1, keepdims=True)


def _prep1(m3x, m3n, bn3):
    rb = P
    vspec = pl.BlockSpec((1, 64), lambda g: (0, 0))
    return pl.pallas_call(
        _prep1_body,
        grid=(N // rb,),
        in_specs=[
            pl.BlockSpec((rb, 64), lambda g: (g, 0)),
            pl.BlockSpec((rb, 64), lambda g: (g, 0)),
            vspec, vspec, vspec, vspec,
        ],
        out_specs=[
            pl.BlockSpec((rb, 64), lambda g: (g, 0)),
            pl.BlockSpec((rb, 64), lambda g: (g, 0)),
            pl.BlockSpec((rb, 1), lambda g: (g, 0)),
        ],
        out_shape=[
            jax.ShapeDtypeStruct((N, 64), jnp.float32),
            jax.ShapeDtypeStruct((N, 64), BF),
            jax.ShapeDtypeStruct((N, 1), jnp.float32),
        ],
    )(m3x, m3n, bn3[0], bn3[1], bn3[2], bn3[3])


def _ec2_body(g_ref, xi_ref, w_ref, b_ref, s_ref, q_ref, mx_ref, mn_ref):
    xi = xi_ref[...]
    tot_s = jnp.zeros((1, 128), jnp.float32)
    tot_q = jnp.zeros((1, 128), jnp.float32)
    mx = None
    mn = None
    for k in range(K):
        xj = g_ref[k]
        h0 = jnp.concatenate([xi, xj - xi], axis=1).astype(BF)
        h = jnp.maximum(
            jnp.dot(h0, w_ref[...], preferred_element_type=jnp.float32)
            + b_ref[...], 0.0)
        tot_s += jnp.sum(h, axis=0, keepdims=True)
        tot_q += jnp.sum(h * h, axis=0, keepdims=True)
        mx = h if mx is None else jnp.maximum(mx, h)
        mn = h if mn is None else jnp.minimum(mn, h)

    @pl.when(pl.program_id(0) == 0)
    def _():
        s_ref[...] = jnp.zeros_like(s_ref)
        q_ref[...] = jnp.zeros_like(q_ref)

    s_ref[...] += tot_s
    q_ref[...] += tot_q
    mx_ref[...] = mx
    mn_ref[...] = mn


def _ec2(g2, x1, w16, b, rb=256):
    stat = jax.ShapeDtypeStruct((1, 128), jnp.float32)
    return pl.pallas_call(
        _ec2_body,
        grid=(N // rb,),
        in_specs=[
            pl.BlockSpec((K, rb, 64), lambda g: (0, g, 0)),
            pl.BlockSpec((rb, 64), lambda g: (g, 0)),
            pl.BlockSpec((128, 128), lambda g: (0, 0)),
            pl.BlockSpec((1, 128), lambda g: (0, 0)),
        ],
        out_specs=[
            pl.BlockSpec((1, 128), lambda g: (0, 0)),
            pl.BlockSpec((1, 128), lambda g: (0, 0)),
            pl.BlockSpec((rb, 128), lambda g: (g, 0)),
            pl.BlockSpec((rb, 128), lambda g: (g, 0)),
        ],
        out_shape=[stat, stat,
                   jax.ShapeDtypeStruct((N, 128), jnp.float32),
                   jax.ShapeDtypeStruct((N, 128), jnp.float32)],
    )(g2, x1, w16, b)


def _l1_body(x1_ref, mx2_ref, mn2_ref, m_ref, sq_ref, g_ref, be_ref,
             w_ref, b_ref, s_ref, q_ref, mx_ref, mn_ref):
    g = g_ref[...]
    sel = jnp.where(g > 0, mx2_ref[...], mn2_ref[...])
    x2 = _bn_stats_apply(sel, m_ref[...], sq_ref[...], float(E), g,
                         be_ref[...])
    h = jnp.concatenate([x1_ref[...], x2], axis=1).astype(BF)
    a = jnp.maximum(
        jnp.dot(h, w_ref[...], preferred_element_type=jnp.float32)
        + b_ref[...], 0.0)
    bsum = jnp.sum(a, axis=0, keepdims=True)
    bsq = jnp.sum(a * a, axis=0, keepdims=True)
    bmax3 = jnp.max(a, axis=0, keepdims=True).reshape(1, 1, 1024)
    bmin3 = jnp.min(a, axis=0, keepdims=True).reshape(1, 1, 1024)
    c = pl.program_id(0)
    r = pl.program_id(1)

    @pl.when(jnp.logical_and(c == 0, r == 0))
    def _():
        s_ref[...] = jnp.zeros_like(s_ref)
        q_ref[...] = jnp.zeros_like(q_ref)

    s_ref[...] += bsum
    q_ref[...] += bsq

    @pl.when(r == 0)
    def _():
        mx_ref[...] = bmax3
        mn_ref[...] = bmin3

    @pl.when(r > 0)
    def _():
        mx_ref[...] = jnp.maximum(mx_ref[...], bmax3)
        mn_ref[...] = jnp.minimum(mn_ref[...], bmin3)


def _l1(x1, mx2, mn2, bnc2, w16, b, rb=256):
    stat = jax.ShapeDtypeStruct((1, 1024), jnp.float32)
    v128 = pl.BlockSpec((1, 128), lambda c, g: (0, 0))
    return pl.pallas_call(
        _l1_body,
        grid=(NB, P // rb),
        in_specs=[
            pl.BlockSpec((rb, 64), lambda c, g: (c * (P // rb) + g, 0)),
            pl.BlockSpec((rb, 128), lambda c, g: (c * (P // rb) + g, 0)),
            pl.BlockSpec((rb, 128), lambda c, g: (c * (P // rb) + g, 0)),
            v128, v128, v128, v128,
            pl.BlockSpec((192, 1024), lambda c, g: (0, 0)),
            pl.BlockSpec((1, 1024), lambda c, g: (0, 0)),
        ],
        out_specs=[
            pl.BlockSpec((1, 1024), lambda c, g: (0, 0)),
            pl.BlockSpec((1, 1024), lambda c, g: (0, 0)),
            pl.BlockSpec((1, 1, 1024), lambda c, g: (c, 0, 0)),
            pl.BlockSpec((1, 1, 1024), lambda c, g: (c, 0, 0)),
        ],
        out_shape=[stat, stat,
                   jax.ShapeDtypeStruct((NB, 1, 1024), jnp.float32),
                   jax.ShapeDtypeStruct((NB, 1, 1024), jnp.float32)],
    )(x1, mx2, mn2, bnc2[0], bnc2[1], bnc2[2], bnc2[3], w16, b)


def _head_body(mx_ref, mn_ref, m_ref, sq_ref, g_ref, be_ref,
               m1w_ref, m1b_ref, m1g_ref, m1be_ref,
               m2w_ref, m2b_ref, m2g_ref, m2be_ref,
               m3w_ref, m3b_ref, out_ref):
    g = g_ref[...]
    sel = jnp.where(g > 0, mx_ref[...], mn_ref[...])
    pooled = _bn_stats_apply(sel, m_ref[...], sq_ref[...], float(N), g,
                             be_ref[...])

    def blk(x, w, bb, gg, be):
        a = jnp.maximum(
            jnp.dot(x.astype(BF), w, preferred_element_type=jnp.float32)
            + bb, 0.0)
        m = jnp.mean(a, axis=0, keepdims=True)
        v = jnp.mean((a - m) * (a - m), axis=0, keepdims=True)
        return gg * (a - m) / jnp.sqrt(v + EPS) + be

    h = blk(pooled, m1w_ref[...], m1b_ref[...], m1g_ref[...], m1be_ref[...])
    h = blk(h, m2w_ref[...], m2b_ref[...], m2g_ref[...], m2be_ref[...])
    logits = jnp.dot(h.astype(BF), m3w_ref[...],
                     preferred_element_type=jnp.float32) + m3b_ref[...]
    zmax = jnp.max(logits, axis=1, keepdims=True)
    shifted = logits - zmax
    lse = jnp.log(jnp.sum(jnp.exp(shifted), axis=1, keepdims=True))
    out_ref[...] = shifted - lse


def _head(mx, mn, bnl, m1w, m1b, m1g, m1be, m2w, m2b, m2g, m2be, m3w, m3b):
    full = lambda s: pl.BlockSpec(s, lambda: (0,) * len(s))
    return pl.pallas_call(
        _head_body,
        grid=(),
        in_specs=[
            full((NB, 1024)), full((NB, 1024)),
            full((1, 1024)), full((1, 1024)), full((1, 1024)),
            full((1, 1024)),
            full((1024, 512)), full((1, 512)), full((1, 512)), full((1, 512)),
            full((512, 256)), full((1, 256)), full((1, 256)), full((1, 256)),
            full((256, 40)), full((1, 40)),
        ],
        out_specs=full((NB, 40)),
        out_shape=jax.ShapeDtypeStruct((NB, 40), jnp.float32),
    )(mx, mn, bnl[0], bnl[1], bnl[2], bnl[3],
      m1w, m1b, m1g, m1be, m2w, m2b, m2g, m2be, m3w, m3b)


def _row(x):
    return x.reshape(1, -1)


def kernel(pos, batch, c1l1w, c1l1b, c1l1g, c1l1be, c1l2w, c1l2b, c1l2g,
           c1l2be, c1l3w, c1l3b, c1l3g, c1l3be, c2l1w, c2l1b, c2l1g, c2l1be,
           l1w, l1b, l1g, l1be, m1w, m1b, m1g, m1be, m2w, m2b, m2g, m2be,
           m3w, m3b):
    f32 = jnp.float32
    ne = float(E)

    w116 = jnp.zeros((8, 64), f32).at[:6].set(c1l1w).astype(BF)
    w216 = c1l2w.astype(BF)
    w316 = c1l3w.astype(BF)
    wc16 = c2l1w.astype(BF)
    wl16 = l1w.astype(BF)

    zb = (_row(jnp.zeros(64, f32)),) * 4

    posp, p16, n2a = _prep0(pos)
    n2arow = n2a.reshape(N)[None, :]

    idx1 = _knn(p16, n2a, n2arow, 8)
    g1 = _gather_rows(posp, idx1.reshape(-1), 16, 2048).reshape(K, N, 16)

    s1, q1 = _ec1(1, g1, posp, w116, _row(c1l1b), zb, w216, _row(c1l2b),
                  zb, w316, _row(c1l3b))[:2]
    bn1 = (s1, q1, _row(c1l1g), _row(c1l1be))
    s2, q2 = _ec1(2, g1, posp, w116, _row(c1l1b), bn1, w216, _row(c1l2b),
                  zb, w316, _row(c1l3b))[:2]
    bn2 = (s2, q2, _row(c1l2g), _row(c1l2be))
    s3, q3, m3x, m3n = _ec1(3, g1, posp, w116, _row(c1l1b), bn1, w216,
                            _row(c1l2b), bn2, w316, _row(c1l3b))
    bn3 = (s3, q3, _row(c1l3g), _row(c1l3be))

    x1, x116, n2b = _prep1(m3x, m3n, bn3)
    n2brow = n2b.reshape(N)[None, :]

    idx2 = _knn(x116, n2b, n2brow, 64)
    g2 = _gather_rows(x1, idx2.reshape(-1), 64, 640).reshape(K, N, 64)

    sc_, qc_, m2x, m2n = _ec2(g2, x1, wc16, _row(c2l1b))
    bnc2 = (sc_, qc_, _row(c2l1g), _row(c2l1be))

    sl_, ql_, mxp, mnp = _l1(x1, m2x, m2n, bnc2, wl16, _row(l1b))
    mxp = mxp.reshape(NB, 1024)
    mnp = mnp.reshape(NB, 1024)
    bnl = (sl_, ql_, _row(l1g), _row(l1be))

    return _head(mxp, mnp, bnl,
                 m1w.astype(BF), _row(m1b), _row(m1g), _row(m1be),
                 m2w.astype(BF), _row(m2b), _row(m2g), _row(m2be),
                 m3w.astype(BF), _row(m3b))
